# baseline (speedup 1.0000x reference)
import functools

import jax
import jax.numpy as jnp
from jax import lax
from jax.experimental import pallas as pl
from jax.experimental.pallas import tpu as pltpu
from jax.experimental.pallas import tpu_sc as plsc

B = 4096
F = 26
K = 64
H = 256
N_LOOKUPS = B * F

_NC = 2
_NS = 16
_NW = _NC * _NS
_BPW = N_LOOKUPS // _NW
_CHUNK = 832
_NCHUNK = _BPW // _CHUNK


def _gather_body(x_hbm, vtab_hbm, wtab_hbm, vout_hbm, wout_hbm,
                 idx_v, vbuf0, vbuf1, wbuf, gsem0, gsem1, wsem):
    wid = lax.axis_index("s") * _NC + lax.axis_index("c")
    base = wid * _BPW
    pltpu.sync_copy(x_hbm.at[pl.ds(base, _BPW)], idx_v)
    wcp = pltpu.async_copy(wtab_hbm.at[idx_v], wbuf, wsem)
    bufs = (vbuf0, vbuf1)
    sems = (gsem0, gsem1)
    cps = [None, None]
    cps[0] = pltpu.async_copy(
        vtab_hbm.at[idx_v.at[pl.ds(0, _CHUNK)]], bufs[0], sems[0])
    for c in range(_NCHUNK):
        cur = c % 2
        nxt = (c + 1) % 2
        if c + 1 < _NCHUNK:
            cps[nxt] = pltpu.async_copy(
                vtab_hbm.at[idx_v.at[pl.ds((c + 1) * _CHUNK, _CHUNK)]],
                bufs[nxt], sems[nxt])
        cps[cur].wait()
        pltpu.sync_copy(bufs[cur],
                        vout_hbm.at[pl.ds(base + c * _CHUNK, _CHUNK)])
    wcp.wait()
    pltpu.sync_copy(wbuf, wout_hbm.at[pl.ds(base, _BPW)])


@functools.cache
def _make_gather():
    return pl.kernel(
        _gather_body,
        out_type=(
            jax.ShapeDtypeStruct((N_LOOKUPS, K), jnp.float32),
            jax.ShapeDtypeStruct((N_LOOKUPS,), jnp.float32),
        ),
        mesh=plsc.VectorSubcoreMesh(core_axis_name="c",
                                    subcore_axis_name="s"),
        compiler_params=pltpu.CompilerParams(use_tc_tiling_on_sc=False),
        scratch_types=(
            pltpu.VMEM((_BPW,), jnp.int32),
            pltpu.VMEM((_CHUNK, K), jnp.float32),
            pltpu.VMEM((_CHUNK, K), jnp.float32),
            pltpu.VMEM((_BPW,), jnp.float32),
            pltpu.SemaphoreType.DMA,
            pltpu.SemaphoreType.DMA,
            pltpu.SemaphoreType.DMA,
        ),
    )

_BB = 2048


def _mlp_body(*refs):
    v_ref, wv_ref = refs[0], refs[1]
    w_refs = refs[2:2 + F]
    ball_ref, clfh_ref, scal_ref = refs[2 + F], refs[3 + F], refs[4 + F]
    out_ref = refs[5 + F]
    v = v_ref[...]
    s2 = v[:, 0:128] * 1.0
    ss2 = v[:, 0:128] * v[:, 0:128]
    for p in range(1, F // 2):
        c = v[:, p * 128:(p + 1) * 128]
        s2 = s2 + c
        ss2 = ss2 + c * c
    s = s2[:, :K] + s2[:, K:]
    ss = ss2[:, :K] + ss2[:, K:]
    fm2 = 0.5 * jnp.sum(s * s - ss, axis=1, keepdims=True)
    fm1 = jnp.sum(wv_ref[...], axis=1, keepdims=True)
    w0 = scal_ref[0]
    clf_w0 = scal_ref[1]
    clf_b = scal_ref[2]
    fm_out = w0 + fm1 + fm2
    h = v
    for i in range(F):
        h = jnp.maximum(
            jnp.dot(h, w_refs[i][...], preferred_element_type=jnp.float32)
            + ball_ref[i, :], 0.0)
    out_ref[...] = (fm_out * clf_w0
                    + jnp.dot(h, clfh_ref[...],
                              preferred_element_type=jnp.float32)
                    + clf_b)


def _deepfm_tc(v2, wv, ws, ball, clfh, scal):
    grid = (B // _BB,)
    w_specs = [pl.BlockSpec(w.shape, lambda i: (0, 0)) for w in ws]
    return pl.pallas_call(
        _mlp_body,
        grid=grid,
        in_specs=[
            pl.BlockSpec((_BB, F * K), lambda i: (i, 0)),
            pl.BlockSpec((_BB, F), lambda i: (i, 0)),
            *w_specs,
            pl.BlockSpec((F, H), lambda i: (0, 0)),
            pl.BlockSpec((H, 1), lambda i: (0, 0)),
            pl.BlockSpec(memory_space=pltpu.SMEM),
        ],
        out_specs=pl.BlockSpec((_BB, 1), lambda i: (i, 0)),
        out_shape=jax.ShapeDtypeStruct((B, 1), jnp.float32),
    )(v2, wv, *ws, ball, clfh, scal)


def kernel(x, W0, W_tab, V_tab, mlp_ws, mlp_bs, clf_w, clf_b):
    x_flat = x.reshape(-1).astype(jnp.int32)
    v_flat, w_flat = _make_gather()(x_flat, V_tab, W_tab.reshape(-1))
    v2 = v_flat.reshape(B, F * K)
    wv = w_flat.reshape(B, F)
    ball = jnp.stack(mlp_bs)
    clfh = clf_w[1:]
    scal = jnp.concatenate([W0.reshape(-1), clf_w[0].reshape(-1),
                            clf_b.reshape(-1)])
    return _deepfm_tc(v2, wv, list(mlp_ws), ball, clfh, scal)


# speedup vs baseline: 1.0093x; 1.0093x over previous
import functools

import jax
import jax.numpy as jnp
from jax import lax
from jax.experimental import pallas as pl
from jax.experimental.pallas import tpu as pltpu
from jax.experimental.pallas import tpu_sc as plsc

B = 4096
F = 26
K = 64
H = 256
N_LOOKUPS = B * F

_NC = 2
_NS = 16
_NW = _NC * _NS
_BPW = N_LOOKUPS // _NW
_CHUNK = 832
_NCHUNK = _BPW // _CHUNK


def _gather_body(x_hbm, vtab_hbm, wtab_hbm, vout_hbm, wout_hbm,
                 idx_v, vbuf0, vbuf1, wbuf, gsem0, gsem1, wsem):
    wid = lax.axis_index("s") * _NC + lax.axis_index("c")
    base = wid * _BPW
    pltpu.sync_copy(x_hbm.at[pl.ds(base, _BPW)], idx_v)
    wcp = pltpu.async_copy(wtab_hbm.at[idx_v], wbuf, wsem)
    bufs = (vbuf0, vbuf1)
    sems = (gsem0, gsem1)
    cps = [None, None]
    cps[0] = pltpu.async_copy(
        vtab_hbm.at[idx_v.at[pl.ds(0, _CHUNK)]], bufs[0], sems[0])
    for c in range(_NCHUNK):
        cur = c % 2
        nxt = (c + 1) % 2
        if c + 1 < _NCHUNK:
            cps[nxt] = pltpu.async_copy(
                vtab_hbm.at[idx_v.at[pl.ds((c + 1) * _CHUNK, _CHUNK)]],
                bufs[nxt], sems[nxt])
        cps[cur].wait()
        pltpu.sync_copy(bufs[cur],
                        vout_hbm.at[pl.ds(base + c * _CHUNK, _CHUNK)])
    wcp.wait()
    pltpu.sync_copy(wbuf, wout_hbm.at[pl.ds(base, _BPW)])


@functools.cache
def _make_gather():
    return pl.kernel(
        _gather_body,
        out_type=(
            jax.ShapeDtypeStruct((N_LOOKUPS, K), jnp.float32),
            jax.ShapeDtypeStruct((N_LOOKUPS,), jnp.float32),
        ),
        mesh=plsc.VectorSubcoreMesh(core_axis_name="c",
                                    subcore_axis_name="s"),
        compiler_params=pltpu.CompilerParams(use_tc_tiling_on_sc=False),
        scratch_types=(
            pltpu.VMEM((_BPW,), jnp.int32),
            pltpu.VMEM((_CHUNK, K), jnp.float32),
            pltpu.VMEM((_CHUNK, K), jnp.float32),
            pltpu.VMEM((_BPW,), jnp.float32),
            pltpu.SemaphoreType.DMA,
            pltpu.SemaphoreType.DMA,
            pltpu.SemaphoreType.DMA,
        ),
    )

_BB = 1024


def _mlp_body(*refs):
    v_ref, wv_ref = refs[0], refs[1]
    w_refs = refs[2:2 + F]
    ball_ref, clfh_ref, scal_ref = refs[2 + F], refs[3 + F], refs[4 + F]
    out_ref = refs[5 + F]
    v = v_ref[...]
    s2 = v[:, 0:128] * 1.0
    ss2 = v[:, 0:128] * v[:, 0:128]
    for p in range(1, F // 2):
        c = v[:, p * 128:(p + 1) * 128]
        s2 = s2 + c
        ss2 = ss2 + c * c
    s = s2[:, :K] + s2[:, K:]
    ss = ss2[:, :K] + ss2[:, K:]
    fm2 = 0.5 * jnp.sum(s * s - ss, axis=1, keepdims=True)
    fm1 = jnp.sum(wv_ref[...], axis=1, keepdims=True)
    w0 = scal_ref[0]
    clf_w0 = scal_ref[1]
    clf_b = scal_ref[2]
    fm_out = w0 + fm1 + fm2
    h = v.astype(jnp.bfloat16)
    for i in range(F):
        h = jnp.maximum(
            jnp.dot(h, w_refs[i][...].astype(jnp.bfloat16),
                    preferred_element_type=jnp.float32)
            + ball_ref[i, :], 0.0).astype(jnp.bfloat16)
    out_ref[...] = (fm_out * clf_w0
                    + jnp.dot(h.astype(jnp.float32), clfh_ref[...],
                              preferred_element_type=jnp.float32)
                    + clf_b)


def _deepfm_tc(v2, wv, ws, ball, clfh, scal):
    grid = (B // _BB,)
    w_specs = [pl.BlockSpec(w.shape, lambda i: (0, 0)) for w in ws]
    return pl.pallas_call(
        _mlp_body,
        grid=grid,
        in_specs=[
            pl.BlockSpec((_BB, F * K), lambda i: (i, 0)),
            pl.BlockSpec((_BB, F), lambda i: (i, 0)),
            *w_specs,
            pl.BlockSpec((F, H), lambda i: (0, 0)),
            pl.BlockSpec((H, 1), lambda i: (0, 0)),
            pl.BlockSpec(memory_space=pltpu.SMEM),
        ],
        out_specs=pl.BlockSpec((_BB, 1), lambda i: (i, 0)),
        out_shape=jax.ShapeDtypeStruct((B, 1), jnp.float32),
    )(v2, wv, *ws, ball, clfh, scal)


def kernel(x, W0, W_tab, V_tab, mlp_ws, mlp_bs, clf_w, clf_b):
    x_flat = x.reshape(-1).astype(jnp.int32)
    v_flat, w_flat = _make_gather()(x_flat, V_tab, W_tab.reshape(-1))
    v2 = v_flat.reshape(B, F * K)
    wv = w_flat.reshape(B, F)
    ball = jnp.stack(mlp_bs)
    clfh = clf_w[1:]
    scal = jnp.concatenate([W0.reshape(-1), clf_w[0].reshape(-1),
                            clf_b.reshape(-1)])
    return _deepfm_tc(v2, wv, list(mlp_ws), ball, clfh, scal)


# speedup vs baseline: 1.1877x; 1.1768x over previous
import functools

import jax
import jax.numpy as jnp
from jax import lax
from jax.experimental import pallas as pl
from jax.experimental.pallas import tpu as pltpu
from jax.experimental.pallas import tpu_sc as plsc

B = 4096
F = 26
K = 64
H = 256
P = F // 2
N_LOOKUPS = B * F
N_PAIRS = B * P

_NW = 32
_PPW = N_PAIRS // _NW
_CHUNK = 416
_NCHUNK = _PPW // _CHUNK
_WPW = N_LOOKUPS // _NW


def _gather_body(idxa_hbm, idxb_hbm, tab_hbm, wtab_hbm, xflat_hbm,
                 vout_hbm, wout_hbm,
                 idxa_v, idxb_v, bufa0, bufb0, bufa1, bufb1, widx_v, wbuf,
                 semA0, semB0, semA1, semB1, wsem):
    wid = lax.axis_index("s") * 2 + lax.axis_index("c")
    pbase = wid * _PPW
    wbase = wid * _WPW
    pltpu.sync_copy(idxa_hbm.at[pl.ds(pbase, _PPW)], idxa_v)
    pltpu.sync_copy(idxb_hbm.at[pl.ds(pbase, _PPW)], idxb_v)
    pltpu.sync_copy(xflat_hbm.at[pl.ds(wbase, _WPW)], widx_v)
    wcp = pltpu.async_copy(wtab_hbm.at[widx_v], wbuf, wsem)
    bufsa = (bufa0, bufa1)
    bufsb = (bufb0, bufb1)
    semsA = (semA0, semA1)
    semsB = (semB0, semB1)
    cps = [None, None]

    def start(c, slot):
        a = pltpu.async_copy(
            tab_hbm.at[idxa_v.at[pl.ds(c * _CHUNK, _CHUNK)]],
            bufsa[slot], semsA[slot])
        b = pltpu.async_copy(
            tab_hbm.at[idxb_v.at[pl.ds(c * _CHUNK, _CHUNK)]],
            bufsb[slot], semsB[slot])
        return (a, b)

    cps[0] = start(0, 0)
    for c in range(_NCHUNK):
        cur = c % 2
        nxt = (c + 1) % 2
        if c + 1 < _NCHUNK:
            cps[nxt] = start(c + 1, nxt)
        cps[cur][0].wait()
        cps[cur][1].wait()
        rows = pl.ds(pbase + c * _CHUNK, _CHUNK)
        pltpu.sync_copy(bufsa[cur], vout_hbm.at[rows, pl.ds(0, K)])
        pltpu.sync_copy(bufsb[cur], vout_hbm.at[rows, pl.ds(K, K)])
    wcp.wait()
    pltpu.sync_copy(wbuf, wout_hbm.at[pl.ds(wbase, _WPW)])


@functools.cache
def _make_gather():
    return pl.kernel(
        _gather_body,
        out_type=(
            jax.ShapeDtypeStruct((N_PAIRS, 2 * K), jnp.float32),
            jax.ShapeDtypeStruct((N_LOOKUPS,), jnp.float32),
        ),
        mesh=plsc.VectorSubcoreMesh(core_axis_name="c",
                                    subcore_axis_name="s"),
        compiler_params=pltpu.CompilerParams(use_tc_tiling_on_sc=False),
        scratch_types=(
            pltpu.VMEM((_PPW,), jnp.int32),
            pltpu.VMEM((_PPW,), jnp.int32),
            pltpu.VMEM((_CHUNK, K), jnp.float32),
            pltpu.VMEM((_CHUNK, K), jnp.float32),
            pltpu.VMEM((_CHUNK, K), jnp.float32),
            pltpu.VMEM((_CHUNK, K), jnp.float32),
            pltpu.VMEM((_WPW,), jnp.int32),
            pltpu.VMEM((_WPW,), jnp.float32),
            pltpu.SemaphoreType.DMA,
            pltpu.SemaphoreType.DMA,
            pltpu.SemaphoreType.DMA,
            pltpu.SemaphoreType.DMA,
            pltpu.SemaphoreType.DMA,
        ),
    )


_BB = 1024


def _mlp_body(*refs):
    v_ref, wv_ref = refs[0], refs[1]
    w_refs = refs[2:2 + F]
    ball_ref, clfh_ref, scal_ref = refs[2 + F], refs[3 + F], refs[4 + F]
    out_ref = refs[5 + F]
    v4 = v_ref[...]
    s128 = v4[0] * 1.0
    ss128 = v4[0] * v4[0]
    for p in range(1, P):
        c = v4[p]
        s128 = s128 + c
        ss128 = ss128 + c * c
    s64 = s128[:, :K] + s128[:, K:]
    fm2 = 0.5 * (jnp.sum(s64 * s64, axis=1, keepdims=True)
                 - jnp.sum(ss128, axis=1, keepdims=True))
    fm1 = jnp.sum(wv_ref[...], axis=1, keepdims=True)
    w0 = scal_ref[0]
    clf_w0 = scal_ref[1]
    clf_b = scal_ref[2]
    fm_out = w0 + fm1 + fm2
    w1_ref = w_refs[0]
    h = jnp.dot(v4[0], w1_ref[0:128, :], preferred_element_type=jnp.float32)
    for p in range(1, P):
        h = h + jnp.dot(v4[p], w1_ref[p * 128:(p + 1) * 128, :],
                        preferred_element_type=jnp.float32)
    h = jnp.maximum(h + ball_ref[0, :], 0.0)
    for i in range(1, F):
        h = jnp.maximum(
            jnp.dot(h, w_refs[i][...], preferred_element_type=jnp.float32)
            + ball_ref[i, :], 0.0)
    out_ref[...] = (fm_out * clf_w0
                    + jnp.dot(h, clfh_ref[...],
                              preferred_element_type=jnp.float32)
                    + clf_b)


def _deepfm_tc(v3, wv, ws, ball, clfh, scal):
    grid = (B // _BB,)
    w_specs = [pl.BlockSpec(w.shape, lambda i: (0, 0)) for w in ws]
    return pl.pallas_call(
        _mlp_body,
        grid=grid,
        in_specs=[
            pl.BlockSpec((P, _BB, 2 * K), lambda i: (0, i, 0)),
            pl.BlockSpec((_BB, F), lambda i: (i, 0)),
            *w_specs,
            pl.BlockSpec((F, H), lambda i: (0, 0)),
            pl.BlockSpec((H, 1), lambda i: (0, 0)),
            pl.BlockSpec(memory_space=pltpu.SMEM),
        ],
        out_specs=pl.BlockSpec((_BB, 1), lambda i: (i, 0)),
        out_shape=jax.ShapeDtypeStruct((B, 1), jnp.float32),
    )(v3, wv, *ws, ball, clfh, scal)


def kernel(x, W0, W_tab, V_tab, mlp_ws, mlp_bs, clf_w, clf_b):
    xi = x.astype(jnp.int32)
    x_flat = xi.reshape(-1)
    xt2 = 2 * xi.T
    idxa = xt2[0::2].reshape(-1)
    idxb = xt2[1::2].reshape(-1)
    tab = jnp.pad(V_tab, ((0, 0), (0, K))).reshape(2 * V_tab.shape[0], K)
    v_pairs, w_flat = _make_gather()(idxa, idxb, tab, W_tab.reshape(-1),
                                     x_flat)
    v3 = v_pairs.reshape(P, B, 2 * K)
    wv = w_flat.reshape(B, F)
    ball = jnp.stack(mlp_bs)
    clfh = clf_w[1:]
    scal = jnp.concatenate([W0.reshape(-1), clf_w[0].reshape(-1),
                            clf_b.reshape(-1)])
    return _deepfm_tc(v3, wv, list(mlp_ws), ball, clfh, scal)
